# baseline (speedup 1.0000x reference)
import jax
import jax.numpy as jnp
from jax.experimental import pallas as pl
from jax.experimental.pallas import tpu as pltpu
from jax.experimental.pallas import tpu_sc as plsc

F = 128
NF = 16
MM = 32
HID = 128
MAXG = 512
EPS = 1e-5


def _sc_gather(table, idx_flat, window):
    k = idx_flat.shape[1]
    c = table.shape[1]
    mesh = plsc.VectorSubcoreMesh(core_axis_name="core", subcore_axis_name="subcore")

    @pl.kernel(out_type=jax.ShapeDtypeStruct((k, c), table.dtype), mesh=mesh)
    def gk(x_hbm, i_hbm, o_hbm):
        def body(i_vmem, o_vmem):
            pltpu.sync_copy(x_hbm.at[i_vmem.at[0]], o_vmem)

        pltpu.emit_pipeline(
            body,
            grid=(k // window,),
            in_specs=[pl.BlockSpec((1, window), lambda i: (0, i))],
            out_specs=[pl.BlockSpec((window, c), lambda i: (i, 0))],
            core_axis_name=("core", "subcore"),
            dimension_semantics=(pltpu.PARALLEL,),
        )(i_hbm, o_hbm)

    return gk(table, idx_flat)


def _conv_pass1(x16, xg16, nbrf16, Wc, Ws, b, ab, off):
    grid = xg16.shape[0] // (ab * MM)
    eb = ab * MM

    def body(x_ref, xg_ref, nf_ref, wc_ref, ws_ref, b_ref, s1_ref, s2_ref):
        i = pl.program_id(0)
        u = jnp.dot(x_ref[...], ws_ref[...],
                    preferred_element_type=jnp.float32) + b_ref[...]
        cat = jnp.concatenate([xg_ref[...].astype(jnp.bfloat16),
                               nf_ref[...]], axis=1)
        ve = jnp.dot(cat, wc_ref[...], preferred_element_type=jnp.float32)
        g3 = ve.reshape(ab, MM, 2 * F) + u[:, None, :]
        s1 = jnp.sum(g3, axis=(0, 1)).reshape(1, 2 * F)
        s2 = jnp.sum(g3 * g3, axis=(0, 1)).reshape(1, 2 * F)

        @pl.when(i == 0)
        def _():
            s1_ref[...] = jnp.zeros_like(s1_ref)
            s2_ref[...] = jnp.zeros_like(s2_ref)

        s1_ref[...] += jnp.broadcast_to(s1, (8, 2 * F))
        s2_ref[...] += jnp.broadcast_to(s2, (8, 2 * F))

    return pl.pallas_call(
        body,
        grid=(grid,),
        in_specs=[
            pl.BlockSpec((ab, F), lambda i, o=off: (i + o, 0)),
            pl.BlockSpec((eb, F), lambda i: (i, 0)),
            pl.BlockSpec((eb, NF), lambda i, o=off: (i + o, 0)),
            pl.BlockSpec((F + NF, 2 * F), lambda i: (0, 0)),
            pl.BlockSpec((F, 2 * F), lambda i: (0, 0)),
            pl.BlockSpec((1, 2 * F), lambda i: (0, 0)),
        ],
        out_specs=[
            pl.BlockSpec((8, 2 * F), lambda i: (0, 0)),
            pl.BlockSpec((8, 2 * F), lambda i: (0, 0)),
        ],
        out_shape=[
            jax.ShapeDtypeStruct((8, 2 * F), jnp.float32),
            jax.ShapeDtypeStruct((8, 2 * F), jnp.float32),
        ],
    )(x16, xg16, nbrf16, Wc, Ws, b)


def _finalize(s1, s2, Wc, Ws, b, g1, be1, nm):
    inv = 1.0 / nm

    def body(s1_ref, s2_ref, wc_ref, ws_ref, b_ref, g1_ref, be1_ref,
             wcp_ref, wsp_ref, bp_ref):
        mean = jnp.sum(s1_ref[...], axis=0, keepdims=True) * (inv / 8)
        var = jnp.sum(s2_ref[...], axis=0, keepdims=True) * (inv / 8) - mean * mean
        a = jax.lax.rsqrt(var + EPS) * g1_ref[...]
        wcp_ref[...] = (wc_ref[...].astype(jnp.float32) * a).astype(jnp.bfloat16)
        wsp_ref[...] = (ws_ref[...].astype(jnp.float32) * a).astype(jnp.bfloat16)
        bp_ref[...] = b_ref[...] * a + be1_ref[...] - mean * a

    return pl.pallas_call(
        body,
        out_shape=[
            jax.ShapeDtypeStruct((F + NF, 2 * F), jnp.bfloat16),
            jax.ShapeDtypeStruct((F, 2 * F), jnp.bfloat16),
            jax.ShapeDtypeStruct((1, 2 * F), jnp.float32),
        ],
    )(s1, s2, Wc, Ws, b, g1, be1)


def _conv_pass2(x16, xg16, nbrf16, Wcp, Wsp, bp, sel, ab, off):
    ch = xg16.shape[0] // MM
    grid = ch // ab
    eb = ab * MM

    def body(x_ref, xg_ref, nf_ref, wc_ref, ws_ref, b_ref, sel_ref, out_ref):
        u = (jnp.dot(x_ref[...], ws_ref[...],
                     preferred_element_type=jnp.float32)
             + b_ref[...]).astype(jnp.bfloat16)
        cat = jnp.concatenate([xg_ref[...].astype(jnp.bfloat16),
                               nf_ref[...]], axis=1)
        ve = jnp.dot(cat, wc_ref[...],
                     preferred_element_type=jnp.float32).astype(jnp.bfloat16)
        normed = ve.reshape(ab, MM, 2 * F) + u[:, None, :]
        filt = normed[:, :, :F]
        core = normed[:, :, F:]
        act = (jax.nn.sigmoid(filt) * jax.nn.softplus(core)).reshape(eb, F)
        out_ref[...] = jnp.dot(sel_ref[...], act,
                               preferred_element_type=jnp.float32)

    return pl.pallas_call(
        body,
        grid=(grid,),
        in_specs=[
            pl.BlockSpec((ab, F), lambda i, o=off: (i + o, 0)),
            pl.BlockSpec((eb, F), lambda i: (i, 0)),
            pl.BlockSpec((eb, NF), lambda i, o=off: (i + o, 0)),
            pl.BlockSpec((F + NF, 2 * F), lambda i: (0, 0)),
            pl.BlockSpec((F, 2 * F), lambda i: (0, 0)),
            pl.BlockSpec((1, 2 * F), lambda i: (0, 0)),
            pl.BlockSpec((ab, eb), lambda i: (0, 0)),
        ],
        out_specs=pl.BlockSpec((ab, F), lambda i: (i, 0)),
        out_shape=jax.ShapeDtypeStruct((ch, F), jnp.float32),
    )(x16, xg16, nbrf16, Wcp, Wsp, bp, sel)


def _embed(ids2, emb_pad, ab):
    n = ids2.shape[0]

    def body(id_ref, e_ref, out_ref, out16_ref):
        oh = (jax.lax.broadcasted_iota(jnp.int32, (ab, 128), 1)
              == id_ref[...]).astype(jnp.float32)
        y = jnp.dot(oh, e_ref[...], preferred_element_type=jnp.float32)
        out_ref[...] = y
        out16_ref[...] = y.astype(jnp.bfloat16)

    return pl.pallas_call(
        body,
        grid=(n // ab,),
        in_specs=[
            pl.BlockSpec((ab, 1), lambda i: (i, 0)),
            pl.BlockSpec((128, F), lambda i: (0, 0)),
        ],
        out_specs=[
            pl.BlockSpec((ab, F), lambda i: (i, 0)),
            pl.BlockSpec((ab, F), lambda i: (i, 0)),
        ],
        out_shape=[
            jax.ShapeDtypeStruct((n, F), jnp.float32),
            jax.ShapeDtypeStruct((n, F), jnp.bfloat16),
        ],
    )(ids2, emb_pad)


def _bn2_res(x, s, g2, be2):
    def body(x_ref, s_ref, g2_ref, be2_ref, out_ref, out16_ref):
        sv = s_ref[...]
        m = jnp.mean(sv, axis=0, keepdims=True)
        v = jnp.mean(sv * sv, axis=0, keepdims=True) - m * m
        normed = (sv - m) * jax.lax.rsqrt(v + EPS) * g2_ref[...] + be2_ref[...]
        res = jax.nn.softplus(x_ref[...] + normed)
        out_ref[...] = res
        out16_ref[...] = res.astype(jnp.bfloat16)

    return pl.pallas_call(
        body,
        out_shape=[
            jax.ShapeDtypeStruct(x.shape, jnp.float32),
            jax.ShapeDtypeStruct(x.shape, jnp.bfloat16),
        ],
    )(x, s, g2, be2)


def _fc_pad(x, fc_W, fc_b, bsz, alen):
    def body(x_ref, w_ref, b_ref, out_ref):
        y = jnp.dot(x_ref[...], w_ref[...], preferred_element_type=jnp.float32) + b_ref[...]
        zer = jnp.zeros((MAXG - alen, HID), jnp.float32)
        out_ref[0, :alen, :] = y[:alen]
        out_ref[0, alen:, :] = zer
        out_ref[1, :alen, :] = y[alen:]
        out_ref[1, alen:, :] = zer

    return pl.pallas_call(
        body,
        grid=(bsz // 2,),
        in_specs=[
            pl.BlockSpec((2 * alen, F), lambda i: (i, 0)),
            pl.BlockSpec((F, HID), lambda i: (0, 0)),
            pl.BlockSpec((1, HID), lambda i: (0, 0)),
        ],
        out_specs=pl.BlockSpec((2, MAXG, HID), lambda i: (i, 0, 0)),
        out_shape=jax.ShapeDtypeStruct((bsz, MAXG, HID), jnp.float32),
    )(x, fc_W, fc_b)


def kernel(atom_num, nbr_idx, nbr_fea, crystal_atom_idx, uni_idx, uni_count,
           emb, c0_W, c0_b, c0_g1, c0_be1, c0_g2, c0_be2,
           c1_W, c1_b, c1_g1, c1_be1, c1_g2, c1_be2,
           c2_W, c2_b, c2_g1, c2_be1, c2_g2, c2_be2,
           fc_W, fc_b):
    n, m = nbr_idx.shape
    ab = 400
    chunks = (6000, 4000)

    sel = (jnp.arange(ab)[:, None]
           == jnp.arange(ab * m)[None, :] // m).astype(jnp.bfloat16)
    emb_pad = jnp.concatenate(
        [emb, jnp.zeros((128 - emb.shape[0], F), emb.dtype)], axis=0)
    x, x16 = _embed(atom_num.reshape(n, 1), emb_pad, ab)

    nbr_flat = nbr_idx.reshape(1, n * m)
    nbrf16 = nbr_fea.reshape(n * m, NF).astype(jnp.bfloat16)
    convs = [
        (c0_W, c0_b, c0_g1, c0_be1, c0_g2, c0_be2),
        (c1_W, c1_b, c1_g1, c1_be1, c1_g2, c1_be2),
        (c2_W, c2_b, c2_g1, c2_be1, c2_g2, c2_be2),
    ]
    for W, b, g1, be1, g2, be2 in convs:
        W16 = W.astype(jnp.bfloat16)
        Ws = W16[:F]
        Wc = W16[F:]
        b2 = b.reshape(1, 2 * F)
        g1r = g1.reshape(1, 2 * F)
        be1r = be1.reshape(1, 2 * F)
        xgs, p1s, offs = [], [], []
        off = 0
        for ch in chunks:
            idx_c = jax.lax.slice(nbr_flat, (0, off * m), (1, (off + ch) * m))
            xgs.append(_sc_gather(x, idx_c, 256))
            offs.append(off // ab)
            off += ch
        for xg_c, o in zip(xgs, offs):
            p1s.append(_conv_pass1(x16, xg_c, nbrf16, Wc, Ws, b2, ab, o))
        s1 = jnp.concatenate([p[0] for p in p1s], axis=0)
        s2 = jnp.concatenate([p[1] for p in p1s], axis=0)
        Wcp, Wsp, bp = _finalize(s1, s2, Wc, Ws, b2, g1r, be1r, n * m)
        s = jnp.concatenate(
            [_conv_pass2(x16, xg_c, nbrf16, Wcp, Wsp, bp, sel, ab, o)
             for xg_c, o in zip(xgs, offs)], axis=0)
        x, x16 = _bn2_res(x, s, g2.reshape(1, F), be2.reshape(1, F))

    bsz, alen = crystal_atom_idx.shape
    new_atom_fea = _fc_pad(x, fc_W, fc_b.reshape(1, HID), bsz, alen)
    mask = jnp.broadcast_to(
        (jnp.arange(MAXG)[None, :] >= alen).astype(jnp.int32), (bsz, MAXG))
    return (new_atom_fea, mask)


# speedup vs baseline: 1.0984x; 1.0984x over previous
import jax
import jax.numpy as jnp
from jax.experimental import pallas as pl
from jax.experimental.pallas import tpu as pltpu
from jax.experimental.pallas import tpu_sc as plsc

F = 128
NF = 16
MM = 32
HID = 128
MAXG = 512
EPS = 1e-5


def _sc_gather(table, idx_flat, window):
    k = idx_flat.shape[1]
    c = table.shape[1]
    mesh = plsc.VectorSubcoreMesh(core_axis_name="core", subcore_axis_name="subcore")

    @pl.kernel(out_type=jax.ShapeDtypeStruct((k, c), table.dtype), mesh=mesh)
    def gk(x_hbm, i_hbm, o_hbm):
        def body(i_vmem, o_vmem):
            pltpu.sync_copy(x_hbm.at[i_vmem.at[0]], o_vmem)

        pltpu.emit_pipeline(
            body,
            grid=(k // window,),
            in_specs=[pl.BlockSpec((1, window), lambda i: (0, i))],
            out_specs=[pl.BlockSpec((window, c), lambda i: (i, 0))],
            core_axis_name=("core", "subcore"),
            dimension_semantics=(pltpu.PARALLEL,),
        )(i_hbm, o_hbm)

    return gk(table, idx_flat)


def _conv_pass1(x16, xg16, nbrf16, Wc, Ws, b, ab, off):
    grid = xg16.shape[0] // (ab * MM)
    eb = ab * MM

    def body(x_ref, xg_ref, nf_ref, wc_ref, ws_ref, b_ref, s1_ref, s2_ref):
        i = pl.program_id(0)
        u = jnp.dot(x_ref[...], ws_ref[...],
                    preferred_element_type=jnp.float32) + b_ref[...]
        cat = jnp.concatenate([xg_ref[...].astype(jnp.bfloat16),
                               nf_ref[...]], axis=1)
        ve = jnp.dot(cat, wc_ref[...], preferred_element_type=jnp.float32)
        g3 = ve.reshape(ab, MM, 2 * F) + u[:, None, :]
        s1 = jnp.sum(g3, axis=(0, 1)).reshape(1, 2 * F)
        s2 = jnp.sum(g3 * g3, axis=(0, 1)).reshape(1, 2 * F)

        @pl.when(i == 0)
        def _():
            s1_ref[...] = jnp.zeros_like(s1_ref)
            s2_ref[...] = jnp.zeros_like(s2_ref)

        s1_ref[...] += jnp.broadcast_to(s1, (8, 2 * F))
        s2_ref[...] += jnp.broadcast_to(s2, (8, 2 * F))

    return pl.pallas_call(
        body,
        grid=(grid,),
        in_specs=[
            pl.BlockSpec((ab, F), lambda i, o=off: (i + o, 0)),
            pl.BlockSpec((eb, F), lambda i: (i, 0)),
            pl.BlockSpec((eb, NF), lambda i, o=off: (i + o, 0)),
            pl.BlockSpec((F + NF, 2 * F), lambda i: (0, 0)),
            pl.BlockSpec((F, 2 * F), lambda i: (0, 0)),
            pl.BlockSpec((1, 2 * F), lambda i: (0, 0)),
        ],
        out_specs=[
            pl.BlockSpec((8, 2 * F), lambda i: (0, 0)),
            pl.BlockSpec((8, 2 * F), lambda i: (0, 0)),
        ],
        out_shape=[
            jax.ShapeDtypeStruct((8, 2 * F), jnp.float32),
            jax.ShapeDtypeStruct((8, 2 * F), jnp.float32),
        ],
    )(x16, xg16, nbrf16, Wc, Ws, b)


def _finalize(s1, s2, Wc, Ws, b, g1, be1, nm):
    inv = 1.0 / nm

    def body(s1_ref, s2_ref, wc_ref, ws_ref, b_ref, g1_ref, be1_ref,
             wcp_ref, wsp_ref, bp_ref):
        mean = jnp.sum(s1_ref[...], axis=0, keepdims=True) * (inv / 8)
        var = jnp.sum(s2_ref[...], axis=0, keepdims=True) * (inv / 8) - mean * mean
        a = jax.lax.rsqrt(var + EPS) * g1_ref[...]
        wcp_ref[...] = (wc_ref[...].astype(jnp.float32) * a).astype(jnp.bfloat16)
        wsp_ref[...] = (ws_ref[...].astype(jnp.float32) * a).astype(jnp.bfloat16)
        bp_ref[...] = b_ref[...] * a + be1_ref[...] - mean * a

    return pl.pallas_call(
        body,
        out_shape=[
            jax.ShapeDtypeStruct((F + NF, 2 * F), jnp.bfloat16),
            jax.ShapeDtypeStruct((F, 2 * F), jnp.bfloat16),
            jax.ShapeDtypeStruct((1, 2 * F), jnp.float32),
        ],
    )(s1, s2, Wc, Ws, b, g1, be1)


def _conv_pass2(x16, xg16, nbrf16, Wcp, Wsp, bp, ab, off):
    ch = xg16.shape[0] // MM
    grid = ch // ab
    eb = ab * MM

    def body(x_ref, xg_ref, nf_ref, wc_ref, ws_ref, b_ref, out_ref):
        u = (jnp.dot(x_ref[...], ws_ref[...],
                     preferred_element_type=jnp.float32)
             + b_ref[...]).astype(jnp.bfloat16)
        cat = jnp.concatenate([xg_ref[...].astype(jnp.bfloat16),
                               nf_ref[...]], axis=1)
        ve = jnp.dot(cat, wc_ref[...],
                     preferred_element_type=jnp.float32).astype(jnp.bfloat16)
        normed = ve.reshape(ab, MM, 2 * F) + u[:, None, :]
        filt = normed[:, :, :F]
        core = normed[:, :, F:]
        act = jax.nn.sigmoid(filt) * jax.nn.softplus(core)
        out_ref[...] = jnp.sum(act, axis=1, dtype=jnp.float32)

    return pl.pallas_call(
        body,
        grid=(grid,),
        in_specs=[
            pl.BlockSpec((ab, F), lambda i, o=off: (i + o, 0)),
            pl.BlockSpec((eb, F), lambda i: (i, 0)),
            pl.BlockSpec((eb, NF), lambda i, o=off: (i + o, 0)),
            pl.BlockSpec((F + NF, 2 * F), lambda i: (0, 0)),
            pl.BlockSpec((F, 2 * F), lambda i: (0, 0)),
            pl.BlockSpec((1, 2 * F), lambda i: (0, 0)),
        ],
        out_specs=pl.BlockSpec((ab, F), lambda i: (i, 0)),
        out_shape=jax.ShapeDtypeStruct((ch, F), jnp.float32),
    )(x16, xg16, nbrf16, Wcp, Wsp, bp)


def _embed(ids2, emb_pad, ab):
    n = ids2.shape[0]

    def body(id_ref, e_ref, out_ref, out16_ref):
        oh = (jax.lax.broadcasted_iota(jnp.int32, (ab, 128), 1)
              == id_ref[...]).astype(jnp.float32)
        y = jnp.dot(oh, e_ref[...], preferred_element_type=jnp.float32)
        out_ref[...] = y
        out16_ref[...] = y.astype(jnp.bfloat16)

    return pl.pallas_call(
        body,
        grid=(n // ab,),
        in_specs=[
            pl.BlockSpec((ab, 1), lambda i: (i, 0)),
            pl.BlockSpec((128, F), lambda i: (0, 0)),
        ],
        out_specs=[
            pl.BlockSpec((ab, F), lambda i: (i, 0)),
            pl.BlockSpec((ab, F), lambda i: (i, 0)),
        ],
        out_shape=[
            jax.ShapeDtypeStruct((n, F), jnp.float32),
            jax.ShapeDtypeStruct((n, F), jnp.bfloat16),
        ],
    )(ids2, emb_pad)


def _bn2_res(x, s, g2, be2):
    def body(x_ref, s_ref, g2_ref, be2_ref, out_ref, out16_ref):
        sv = s_ref[...]
        m = jnp.mean(sv, axis=0, keepdims=True)
        v = jnp.mean(sv * sv, axis=0, keepdims=True) - m * m
        normed = (sv - m) * jax.lax.rsqrt(v + EPS) * g2_ref[...] + be2_ref[...]
        res = jax.nn.softplus(x_ref[...] + normed)
        out_ref[...] = res
        out16_ref[...] = res.astype(jnp.bfloat16)

    return pl.pallas_call(
        body,
        out_shape=[
            jax.ShapeDtypeStruct(x.shape, jnp.float32),
            jax.ShapeDtypeStruct(x.shape, jnp.bfloat16),
        ],
    )(x, s, g2, be2)


def _fc_pad(x, fc_W, fc_b, bsz, alen):
    def body(x_ref, w_ref, b_ref, out_ref):
        y = jnp.dot(x_ref[...], w_ref[...], preferred_element_type=jnp.float32) + b_ref[...]
        zer = jnp.zeros((MAXG - alen, HID), jnp.float32)
        out_ref[0, :alen, :] = y[:alen]
        out_ref[0, alen:, :] = zer
        out_ref[1, :alen, :] = y[alen:]
        out_ref[1, alen:, :] = zer

    return pl.pallas_call(
        body,
        grid=(bsz // 2,),
        in_specs=[
            pl.BlockSpec((2 * alen, F), lambda i: (i, 0)),
            pl.BlockSpec((F, HID), lambda i: (0, 0)),
            pl.BlockSpec((1, HID), lambda i: (0, 0)),
        ],
        out_specs=pl.BlockSpec((2, MAXG, HID), lambda i: (i, 0, 0)),
        out_shape=jax.ShapeDtypeStruct((bsz, MAXG, HID), jnp.float32),
    )(x, fc_W, fc_b)


def kernel(atom_num, nbr_idx, nbr_fea, crystal_atom_idx, uni_idx, uni_count,
           emb, c0_W, c0_b, c0_g1, c0_be1, c0_g2, c0_be2,
           c1_W, c1_b, c1_g1, c1_be1, c1_g2, c1_be2,
           c2_W, c2_b, c2_g1, c2_be1, c2_g2, c2_be2,
           fc_W, fc_b):
    n, m = nbr_idx.shape
    ab = 400
    chunks = (6000, 4000)

    emb_pad = jnp.concatenate(
        [emb, jnp.zeros((128 - emb.shape[0], F), emb.dtype)], axis=0)
    x, x16 = _embed(atom_num.reshape(n, 1), emb_pad, ab)

    nbr_flat = nbr_idx.reshape(1, n * m)
    nbrf16 = nbr_fea.reshape(n * m, NF).astype(jnp.bfloat16)
    convs = [
        (c0_W, c0_b, c0_g1, c0_be1, c0_g2, c0_be2),
        (c1_W, c1_b, c1_g1, c1_be1, c1_g2, c1_be2),
        (c2_W, c2_b, c2_g1, c2_be1, c2_g2, c2_be2),
    ]
    for W, b, g1, be1, g2, be2 in convs:
        W16 = W.astype(jnp.bfloat16)
        Ws = W16[:F]
        Wc = W16[F:]
        b2 = b.reshape(1, 2 * F)
        g1r = g1.reshape(1, 2 * F)
        be1r = be1.reshape(1, 2 * F)
        xgs, p1s, offs = [], [], []
        off = 0
        for ch in chunks:
            idx_c = jax.lax.slice(nbr_flat, (0, off * m), (1, (off + ch) * m))
            xgs.append(_sc_gather(x, idx_c, 256))
            offs.append(off // ab)
            off += ch
        for xg_c, o in zip(xgs, offs):
            p1s.append(_conv_pass1(x16, xg_c, nbrf16, Wc, Ws, b2, ab, o))
        s1 = jnp.concatenate([p[0] for p in p1s], axis=0)
        s2 = jnp.concatenate([p[1] for p in p1s], axis=0)
        Wcp, Wsp, bp = _finalize(s1, s2, Wc, Ws, b2, g1r, be1r, n * m)
        s = jnp.concatenate(
            [_conv_pass2(x16, xg_c, nbrf16, Wcp, Wsp, bp, ab, o)
             for xg_c, o in zip(xgs, offs)], axis=0)
        x, x16 = _bn2_res(x, s, g2.reshape(1, F), be2.reshape(1, F))

    bsz, alen = crystal_atom_idx.shape
    new_atom_fea = _fc_pad(x, fc_W, fc_b.reshape(1, HID), bsz, alen)
    mask = jnp.broadcast_to(
        (jnp.arange(MAXG)[None, :] >= alen).astype(jnp.int32), (bsz, MAXG))
    return (new_atom_fea, mask)
